# Initial kernel scaffold; baseline (speedup 1.0000x reference)
import jax
import jax.numpy as jnp
from jax.experimental import pallas as pl


def kernel(signal, hidden, episodic_keys, episodic_values, episodic_strength, episodic_age, Wk, bk, Wv, bv, Wwg, bwg, Wpg, bpg, Wmg, bmg):
    raise NotImplementedError("write your pallas kernel here")


# speedup vs baseline: 1.8814x; 1.8814x over previous
import jax
import jax.numpy as jnp
from jax import lax
from jax.experimental import pallas as pl
from jax.experimental.pallas import tpu as pltpu

_STRENGTH_DECAY = 0.99
_AGE_INCREMENT = 0.02
_TEMPERATURE = 0.1
_EPS = 1e-6

_BLK = 8


def _body(sig_ref, hid_ref, keys_ref, vals_ref, str_ref, age_ref,
          wk_ref, bk_ref, wv_ref, bv_ref, wwg_ref, bwg_ref,
          wpg_ref, bpg_ref, wmg_ref, bmg_ref,
          okeys_ref, ovals_ref, ostr_ref, oage_ref):
    f32 = jnp.float32
    sig = sig_ref[...]
    hid = hid_ref[...]
    joined = jnp.concatenate([sig, hid], axis=-1)

    ck = jnp.tanh(
        lax.dot_general(joined, wk_ref[...], (((1,), (1,)), ((), ())),
                        preferred_element_type=f32) + bk_ref[...])
    cv = jnp.tanh(
        lax.dot_general(joined, wv_ref[...], (((1,), (1,)), ((), ())),
                        preferred_element_type=f32) + bv_ref[...])
    ws = jax.nn.sigmoid(
        jnp.sum(joined * wwg_ref[...], axis=-1, keepdims=True) + bwg_ref[...])
    pg = jax.nn.sigmoid(
        jnp.sum(joined * wpg_ref[...], axis=-1, keepdims=True) + bpg_ref[...])
    mg_lin = jnp.sum(joined * wmg_ref[...], axis=-1, keepdims=True) + bmg_ref[...]

    cnorm = jnp.sqrt(jnp.sum(ck * ck, axis=-1, keepdims=True))
    ncand = ck / jnp.maximum(cnorm, _EPS)

    keys = keys_ref[...]
    keysq = jnp.sum(keys * keys, axis=-1)
    keynorm = jnp.maximum(jnp.sqrt(keysq), _EPS)
    dots = jnp.sum(keys * ncand[:, None, :], axis=-1)
    sim = dots / keynorm

    n = sim.shape[-1]
    iota = lax.broadcasted_iota(jnp.int32, sim.shape, 1)
    neg = jnp.float32(-jnp.inf)

    s1 = jnp.max(sim, axis=-1, keepdims=True)
    i1 = jnp.argmax(sim, axis=-1, keepdims=True)
    sim_m = jnp.where(iota == i1, neg, sim)
    s2 = jnp.max(sim_m, axis=-1, keepdims=True)
    i2 = jnp.argmax(sim_m, axis=-1, keepdims=True)
    sim_m = jnp.where(iota == i2, neg, sim_m)
    s3 = jnp.max(sim_m, axis=-1, keepdims=True)
    i3 = jnp.argmax(sim_m, axis=-1, keepdims=True)

    strength = str_ref[...]
    age = age_ref[...]
    replace_scores = 1.2 * age + 1.0 * (1.0 - strength) + 0.5 * (1.0 - sim)
    ri = jnp.argmax(replace_scores, axis=-1, keepdims=True)

    novelty = jnp.clip(1.0 - s1, 0.0, 1.0)
    merge_pref = jax.nn.sigmoid(mg_lin + 2.6 * s1)
    full_m = (s1 > 0.78) & (merge_pref >= 0.55)
    multi_m = full_m & (s2 > 0.68)
    partial_m = (~multi_m) & (s1 > 0.64) & (s2 > 0.52)

    e2 = jnp.exp((s2 - s1) / _TEMPERATURE)
    e3 = jnp.exp((s3 - s1) / _TEMPERATURE)
    pden = 1.0 + e2
    pw1 = 1.0 / pden
    pw2 = e2 / pden
    mden = 1.0 + e2 + e3
    mw1 = 1.0 / mden
    mw2 = e2 / mden
    mw3 = e3 / mden

    oh1 = (iota == i1).astype(f32)
    oh2 = (iota == i2).astype(f32)
    oh3 = (iota == i3).astype(f32)
    tw = (iota == ri).astype(f32)
    tw = jnp.where(full_m, oh1, tw)
    tw = jnp.where(partial_m, pw1 * oh1 + pw2 * oh2, tw)
    tw = jnp.where(multi_m, mw1 * oh1 + mw2 * oh2 + mw3 * oh3, tw)

    scale = jnp.where(multi_m, 0.16 + 0.52 * ws,
                      jnp.where(partial_m, 0.18 + 0.62 * ws,
                                0.2 + 0.8 * ws))
    ow = tw * (scale * (0.55 + 0.45 * novelty))

    merge_like = full_m | partial_m | multi_m
    kmix = jnp.where(merge_like, 0.28 + 0.24 * pg, 0.78 + 0.16 * pg)
    vmix = jnp.where(merge_like, 0.42 + 0.28 * pg, 0.82 + 0.12 * pg)

    owk = (ow * kmix)[:, :, None]
    okeys_ref[...] = keys + owk * (ck[:, None, :] - keys)
    vals = vals_ref[...]
    owv = (ow * vmix)[:, :, None]
    ovals_ref[...] = vals + owv * (cv[:, None, :] - vals)

    boost = ow * (0.45 + 0.35 * pg + 0.45 * novelty + 0.25 * ws)
    ostr_ref[...] = jnp.clip(strength * _STRENGTH_DECAY + boost, 0.0, 1.0)
    oage_ref[...] = jnp.clip((age + _AGE_INCREMENT) * (1.0 - ow), 0.0, 1.0)


def kernel(signal, hidden, episodic_keys, episodic_values, episodic_strength,
           episodic_age, Wk, bk, Wv, bv, Wwg, bwg, Wpg, bpg, Wmg, bmg,
           interpret=False):
    B, N, D = episodic_keys.shape
    R = _BLK
    grid = (B // R,)

    row = lambda i: (i, 0)
    row3 = lambda i: (i, 0, 0)
    const2 = lambda i: (0, 0)

    in_specs = [
        pl.BlockSpec((R, D), row),
        pl.BlockSpec((R, D), row),
        pl.BlockSpec((R, N, D), row3),
        pl.BlockSpec((R, N, D), row3),
        pl.BlockSpec((R, N), row),
        pl.BlockSpec((R, N), row),
        pl.BlockSpec((D, 2 * D), const2),
        pl.BlockSpec((1, D), const2),
        pl.BlockSpec((D, 2 * D), const2),
        pl.BlockSpec((1, D), const2),
        pl.BlockSpec((1, 2 * D), const2),
        pl.BlockSpec((1, 1), const2),
        pl.BlockSpec((1, 2 * D), const2),
        pl.BlockSpec((1, 1), const2),
        pl.BlockSpec((1, 2 * D), const2),
        pl.BlockSpec((1, 1), const2),
    ]
    out_specs = [
        pl.BlockSpec((R, N, D), row3),
        pl.BlockSpec((R, N, D), row3),
        pl.BlockSpec((R, N), row),
        pl.BlockSpec((R, N), row),
    ]
    out_shapes = [
        jax.ShapeDtypeStruct((B, N, D), jnp.float32),
        jax.ShapeDtypeStruct((B, N, D), jnp.float32),
        jax.ShapeDtypeStruct((B, N), jnp.float32),
        jax.ShapeDtypeStruct((B, N), jnp.float32),
    ]

    out = pl.pallas_call(
        _body,
        grid=grid,
        in_specs=in_specs,
        out_specs=out_specs,
        out_shape=out_shapes,
        compiler_params=pltpu.CompilerParams(
            dimension_semantics=("parallel",)),
        interpret=interpret,
    )(signal, hidden, episodic_keys, episodic_values,
      episodic_strength, episodic_age,
      Wk, bk.reshape(1, D), Wv, bv.reshape(1, D),
      Wwg, bwg.reshape(1, 1), Wpg, bpg.reshape(1, 1), Wmg, bmg.reshape(1, 1))
    return tuple(out)


# speedup vs baseline: 2.0822x; 1.1067x over previous
import jax
import jax.numpy as jnp
from jax import lax
from jax.experimental import pallas as pl
from jax.experimental.pallas import tpu as pltpu

_STRENGTH_DECAY = 0.99
_AGE_INCREMENT = 0.02
_TEMPERATURE = 0.1
_EPS = 1e-6

_BLK = 16


def _body(sig_ref, hid_ref, keys_ref, vals_ref, str_ref, age_ref,
          wk_ref, bk_ref, wv_ref, bv_ref, wwg_ref, bwg_ref,
          wpg_ref, bpg_ref, wmg_ref, bmg_ref,
          okeys_ref, ovals_ref, ostr_ref, oage_ref):
    f32 = jnp.float32
    sig = sig_ref[...]
    hid = hid_ref[...]
    joined = jnp.concatenate([sig, hid], axis=-1)

    ck = jnp.tanh(
        lax.dot_general(joined, wk_ref[...], (((1,), (1,)), ((), ())),
                        preferred_element_type=f32) + bk_ref[...])
    cv = jnp.tanh(
        lax.dot_general(joined, wv_ref[...], (((1,), (1,)), ((), ())),
                        preferred_element_type=f32) + bv_ref[...])
    ws = jax.nn.sigmoid(
        jnp.sum(joined * wwg_ref[...], axis=-1, keepdims=True) + bwg_ref[...])
    pg = jax.nn.sigmoid(
        jnp.sum(joined * wpg_ref[...], axis=-1, keepdims=True) + bpg_ref[...])
    mg_lin = jnp.sum(joined * wmg_ref[...], axis=-1, keepdims=True) + bmg_ref[...]

    cnorm = jnp.sqrt(jnp.sum(ck * ck, axis=-1, keepdims=True))
    ncand = ck / jnp.maximum(cnorm, _EPS)

    keys = keys_ref[...]
    keysq = jnp.sum(keys * keys, axis=-1)
    keynorm = jnp.maximum(jnp.sqrt(keysq), _EPS)
    dots = jnp.sum(keys * ncand[:, None, :], axis=-1)
    sim = dots / keynorm

    n = sim.shape[-1]
    iota = lax.broadcasted_iota(jnp.int32, sim.shape, 1)
    neg = jnp.float32(-jnp.inf)

    s1 = jnp.max(sim, axis=-1, keepdims=True)
    i1 = jnp.argmax(sim, axis=-1, keepdims=True)
    sim_m = jnp.where(iota == i1, neg, sim)
    s2 = jnp.max(sim_m, axis=-1, keepdims=True)
    i2 = jnp.argmax(sim_m, axis=-1, keepdims=True)
    sim_m = jnp.where(iota == i2, neg, sim_m)
    s3 = jnp.max(sim_m, axis=-1, keepdims=True)
    i3 = jnp.argmax(sim_m, axis=-1, keepdims=True)

    strength = str_ref[...]
    age = age_ref[...]
    replace_scores = 1.2 * age + 1.0 * (1.0 - strength) + 0.5 * (1.0 - sim)
    ri = jnp.argmax(replace_scores, axis=-1, keepdims=True)

    novelty = jnp.clip(1.0 - s1, 0.0, 1.0)
    merge_pref = jax.nn.sigmoid(mg_lin + 2.6 * s1)
    full_m = (s1 > 0.78) & (merge_pref >= 0.55)
    multi_m = full_m & (s2 > 0.68)
    partial_m = (~multi_m) & (s1 > 0.64) & (s2 > 0.52)

    e2 = jnp.exp((s2 - s1) / _TEMPERATURE)
    e3 = jnp.exp((s3 - s1) / _TEMPERATURE)
    pden = 1.0 + e2
    pw1 = 1.0 / pden
    pw2 = e2 / pden
    mden = 1.0 + e2 + e3
    mw1 = 1.0 / mden
    mw2 = e2 / mden
    mw3 = e3 / mden

    oh1 = (iota == i1).astype(f32)
    oh2 = (iota == i2).astype(f32)
    oh3 = (iota == i3).astype(f32)
    tw = (iota == ri).astype(f32)
    tw = jnp.where(full_m, oh1, tw)
    tw = jnp.where(partial_m, pw1 * oh1 + pw2 * oh2, tw)
    tw = jnp.where(multi_m, mw1 * oh1 + mw2 * oh2 + mw3 * oh3, tw)

    scale = jnp.where(multi_m, 0.16 + 0.52 * ws,
                      jnp.where(partial_m, 0.18 + 0.62 * ws,
                                0.2 + 0.8 * ws))
    ow = tw * (scale * (0.55 + 0.45 * novelty))

    merge_like = full_m | partial_m | multi_m
    kmix = jnp.where(merge_like, 0.28 + 0.24 * pg, 0.78 + 0.16 * pg)
    vmix = jnp.where(merge_like, 0.42 + 0.28 * pg, 0.82 + 0.12 * pg)

    owk = (ow * kmix)[:, :, None]
    okeys_ref[...] = keys + owk * (ck[:, None, :] - keys)
    vals = vals_ref[...]
    owv = (ow * vmix)[:, :, None]
    ovals_ref[...] = vals + owv * (cv[:, None, :] - vals)

    boost = ow * (0.45 + 0.35 * pg + 0.45 * novelty + 0.25 * ws)
    ostr_ref[...] = jnp.clip(strength * _STRENGTH_DECAY + boost, 0.0, 1.0)
    oage_ref[...] = jnp.clip((age + _AGE_INCREMENT) * (1.0 - ow), 0.0, 1.0)


def kernel(signal, hidden, episodic_keys, episodic_values, episodic_strength,
           episodic_age, Wk, bk, Wv, bv, Wwg, bwg, Wpg, bpg, Wmg, bmg,
           interpret=False):
    B, N, D = episodic_keys.shape
    R = _BLK
    grid = (B // R,)

    row = lambda i: (i, 0)
    row3 = lambda i: (i, 0, 0)
    const2 = lambda i: (0, 0)

    in_specs = [
        pl.BlockSpec((R, D), row),
        pl.BlockSpec((R, D), row),
        pl.BlockSpec((R, N, D), row3),
        pl.BlockSpec((R, N, D), row3),
        pl.BlockSpec((R, N), row),
        pl.BlockSpec((R, N), row),
        pl.BlockSpec((D, 2 * D), const2),
        pl.BlockSpec((1, D), const2),
        pl.BlockSpec((D, 2 * D), const2),
        pl.BlockSpec((1, D), const2),
        pl.BlockSpec((1, 2 * D), const2),
        pl.BlockSpec((1, 1), const2),
        pl.BlockSpec((1, 2 * D), const2),
        pl.BlockSpec((1, 1), const2),
        pl.BlockSpec((1, 2 * D), const2),
        pl.BlockSpec((1, 1), const2),
    ]
    out_specs = [
        pl.BlockSpec((R, N, D), row3),
        pl.BlockSpec((R, N, D), row3),
        pl.BlockSpec((R, N), row),
        pl.BlockSpec((R, N), row),
    ]
    out_shapes = [
        jax.ShapeDtypeStruct((B, N, D), jnp.float32),
        jax.ShapeDtypeStruct((B, N, D), jnp.float32),
        jax.ShapeDtypeStruct((B, N), jnp.float32),
        jax.ShapeDtypeStruct((B, N), jnp.float32),
    ]

    out = pl.pallas_call(
        _body,
        grid=grid,
        in_specs=in_specs,
        out_specs=out_specs,
        out_shape=out_shapes,
        compiler_params=pltpu.CompilerParams(
            dimension_semantics=("parallel",)),
        interpret=interpret,
    )(signal, hidden, episodic_keys, episodic_values,
      episodic_strength, episodic_age,
      Wk, bk.reshape(1, D), Wv, bv.reshape(1, D),
      Wwg, bwg.reshape(1, 1), Wpg, bpg.reshape(1, 1), Wmg, bmg.reshape(1, 1))
    return tuple(out)
